# Initial kernel scaffold; baseline (speedup 1.0000x reference)
import jax
import jax.numpy as jnp
from jax.experimental import pallas as pl


def kernel(userid, poii, Ci, neg_p, UserPreference, PoiPreference):
    raise NotImplementedError("write your pallas kernel here")


# speedup vs baseline: 1.8044x; 1.8044x over previous
import functools

import jax
import jax.numpy as jnp
from jax import lax
from jax.experimental import pallas as pl
from jax.experimental.pallas import tpu as pltpu
from jax.experimental.pallas import tpu_sc as plsc

_NEG = 200
_CI = 200
_D = 128
_ROWS_PER_TILE = 8
_NEG_TILES = _NEG // _ROWS_PER_TILE
_PI_TILE = _NEG_TILES
_U_TILE = _NEG_TILES + 1
_CSUM_BASE = _NEG + _ROWS_PER_TILE
_U_ROW = _CSUM_BASE + _NEG_TILES
_PACK_ROWS = 240


@functools.cache
def _make_sc_gather():
    @functools.partial(
        pl.kernel,
        out_type=jax.ShapeDtypeStruct((_PACK_ROWS, _D), jnp.float32),
        mesh=plsc.VectorSubcoreMesh(core_axis_name="c", subcore_axis_name="s"),
        scratch_types=[
            pltpu.VMEM((_ROWS_PER_TILE,), jnp.int32),
            pltpu.VMEM((_ROWS_PER_TILE,), jnp.int32),
            pltpu.VMEM((_ROWS_PER_TILE, _D), jnp.float32),
            pltpu.VMEM((_ROWS_PER_TILE, _D), jnp.float32),
            pltpu.VMEM((1, _D), jnp.float32),
            pltpu.SemaphoreType.DMA,
            pltpu.SemaphoreType.DMA,
        ],
    )
    def _sc_gather(pidx_hbm, cidx_hbm, uidx_hbm, up_hbm, pp_hbm, out_hbm,
                   nidx, cidx, nrows, crows, srow, sem1, sem2):
        wid = lax.axis_index("s") * 2 + lax.axis_index("c")

        @pl.when(wid < _NEG_TILES)
        def _():
            base = wid * _ROWS_PER_TILE
            pltpu.sync_copy(pidx_hbm.at[pl.ds(base, _ROWS_PER_TILE)], nidx)
            pltpu.sync_copy(cidx_hbm.at[pl.ds(base, _ROWS_PER_TILE)], cidx)
            g1 = pltpu.async_copy(pp_hbm.at[nidx], nrows, sem1)
            g2 = pltpu.async_copy(pp_hbm.at[cidx], crows, sem2)
            g1.wait()
            g2.wait()
            pltpu.sync_copy(nrows, out_hbm.at[pl.ds(base, _ROWS_PER_TILE)])
            for c in range(_D // 16):
                sl = pl.ds(c * 16, 16)
                acc = crows[0, sl]
                for r in range(1, _ROWS_PER_TILE):
                    acc = acc + crows[r, sl]
                srow[0, sl] = acc
            pltpu.sync_copy(srow, out_hbm.at[pl.ds(_CSUM_BASE + wid, 1)])

        @pl.when(wid == _PI_TILE)
        def _():
            pltpu.sync_copy(pidx_hbm.at[pl.ds(_NEG, _ROWS_PER_TILE)], nidx)
            pltpu.async_copy(pp_hbm.at[nidx], nrows, sem1).wait()
            pltpu.sync_copy(nrows, out_hbm.at[pl.ds(_NEG, _ROWS_PER_TILE)])

        @pl.when(wid == _U_TILE)
        def _():
            pltpu.sync_copy(uidx_hbm, cidx)
            pltpu.async_copy(up_hbm.at[cidx], crows, sem2).wait()
            pltpu.sync_copy(crows.at[pl.ds(0, 1)], out_hbm.at[pl.ds(_U_ROW, 1)])

    return _sc_gather


def _finish_body(x_ref, o_ref):
    neg = x_ref[0:_NEG, :]
    pi = x_ref[_NEG:_NEG + 1, :]
    u = x_ref[_U_ROW:_U_ROW + 1, :]
    csum = jnp.sum(x_ref[_CSUM_BASE:_CSUM_BASE + _NEG_TILES, :],
                   axis=0, keepdims=True)

    s = jnp.sum(u * pi)
    t = jnp.sum(csum * pi) / float(_CI)
    a = jnp.sum(neg * u, axis=1, keepdims=True)
    b = jnp.sum(neg * csum, axis=1, keepdims=True) / float(_CI)

    score = jax.nn.sigmoid(s) * jax.nn.sigmoid(t)
    neg_score = jax.nn.sigmoid(a) * jax.nn.sigmoid(b)
    loss = -(jnp.log(score) + jnp.sum(jnp.log(1.0 - neg_score)))
    o_ref[0, 0] = loss


_finish = pl.pallas_call(
    _finish_body,
    out_shape=jax.ShapeDtypeStruct((1, 1), jnp.float32),
    out_specs=pl.BlockSpec(memory_space=pltpu.SMEM),
)


def kernel(userid, poii, Ci, neg_p, UserPreference, PoiPreference):
    pidx = jnp.concatenate(
        [neg_p.astype(jnp.int32),
         jnp.broadcast_to(poii.astype(jnp.int32), (_ROWS_PER_TILE,))])
    cidx = Ci.astype(jnp.int32)
    uidx = jnp.broadcast_to(userid.astype(jnp.int32), (_ROWS_PER_TILE,))
    packed = _make_sc_gather()(pidx, cidx, uidx, UserPreference, PoiPreference)
    return _finish(packed)[0, 0]
